# Initial kernel scaffold; baseline (speedup 1.0000x reference)
import jax
import jax.numpy as jnp
from jax.experimental import pallas as pl


def kernel(loc_ids, time_ids, spatial_table, temporal_table):
    raise NotImplementedError("write your pallas kernel here")


# speedup vs baseline: 3.4630x; 3.4630x over previous
import functools

import jax
import jax.numpy as jnp
from jax import lax
from jax.experimental import pallas as pl
from jax.experimental.pallas import tpu as pltpu
from jax.experimental.pallas import tpu_sc as plsc

NUM_LOCATIONS = 1000000
NUM_TIME_SLOTS = 1440
EMBED_DIM = 64
BATCH = 16384
SEQ = 50

N = BATCH * SEQ
NC, NS = 2, 16
NW = NC * NS
PER_W = N // NW
CHUNK = 128
G = PER_W // CHUNK
D = EMBED_DIM


def _sc_body(loc_hbm, time_hbm, spat_hbm, tmp_hbm, out_hbm,
             loc_v, time_v, spat_v, tmp_v, sem):
    wid = lax.axis_index("s") * NC + lax.axis_index("c")
    w_base = wid * PER_W

    def step(g, carry):
        base = w_base + g * CHUNK
        pltpu.sync_copy(loc_hbm.at[pl.ds(base, CHUNK)], loc_v)
        pltpu.sync_copy(time_hbm.at[pl.ds(base, CHUNK)], time_v)
        cp_s = pltpu.async_copy(spat_hbm.at[loc_v], spat_v, sem)
        cp_t = pltpu.async_copy(tmp_hbm.at[time_v], tmp_v, sem)
        cp_s.wait()
        cp_t.wait()

        def add_row(r, c):
            for j in range(D // 16):
                sl = pl.ds(j * 16, 16)
                spat_v[r, sl] = spat_v[r, sl] + tmp_v[r, sl]
            return c

        lax.fori_loop(0, CHUNK, add_row, 0)
        pltpu.sync_copy(spat_v, out_hbm.at[pl.ds(base, CHUNK)])
        return carry

    lax.fori_loop(0, G, step, 0)


@jax.jit
def _run(loc_flat, time_flat, spatial_table, temporal_table):
    mesh = plsc.VectorSubcoreMesh(core_axis_name="c", subcore_axis_name="s")
    f = pl.kernel(
        _sc_body,
        out_type=jax.ShapeDtypeStruct((N, D), jnp.float32),
        mesh=mesh,
        scratch_types=[
            pltpu.VMEM((CHUNK,), jnp.int32),
            pltpu.VMEM((CHUNK,), jnp.int32),
            pltpu.VMEM((CHUNK, D), jnp.float32),
            pltpu.VMEM((CHUNK, D), jnp.float32),
            pltpu.SemaphoreType.DMA,
        ],
        compiler_params=pltpu.CompilerParams(use_tc_tiling_on_sc=False),
    )
    return f(loc_flat, time_flat, spatial_table, temporal_table)


def kernel(loc_ids, time_ids, spatial_table, temporal_table):
    loc_flat = loc_ids.reshape(-1).astype(jnp.int32)
    time_flat = time_ids.reshape(-1).astype(jnp.int32)
    out = _run(loc_flat, time_flat, spatial_table, temporal_table)
    return out.reshape(BATCH, SEQ, D)


# speedup vs baseline: 4.4229x; 1.2772x over previous
import jax
import jax.numpy as jnp
from jax import lax
from jax.experimental import pallas as pl
from jax.experimental.pallas import tpu as pltpu
from jax.experimental.pallas import tpu_sc as plsc

NUM_LOCATIONS = 1000000
NUM_TIME_SLOTS = 1440
EMBED_DIM = 64
BATCH = 16384
SEQ = 50

N = BATCH * SEQ
NC, NS = 2, 16
NW = NC * NS
PER_W = N // NW
CHUNK = 128
G = PER_W // CHUNK
D = EMBED_DIM
NBUF = 2


def _sc_body(loc_hbm, time_hbm, spat_hbm, tmp_hbm, out_hbm,
             loc_v, time_v, sbuf, tbuf, obuf, *sems):
    sem_g = sems[:NBUF]
    sem_s = sems[NBUF:]
    wid = lax.axis_index("s") * NC + lax.axis_index("c")
    w_base = wid * PER_W

    pltpu.sync_copy(loc_hbm.at[wid], loc_v)
    pltpu.sync_copy(time_hbm.at[wid], time_v)

    def issue_gathers(g, b):
        pltpu.async_copy(spat_hbm.at[loc_v.at[g]], sbuf.at[b], sem_g[b])
        pltpu.async_copy(tmp_hbm.at[time_v.at[g]], tbuf.at[b], sem_g[b])

    def handle(g, b, first):
        pltpu.make_async_copy(spat_hbm.at[loc_v.at[g]], sbuf.at[b], sem_g[b]).wait()
        pltpu.make_async_copy(tmp_hbm.at[time_v.at[g]], tbuf.at[b], sem_g[b]).wait()
        if not first:
            prev = w_base + (g - NBUF) * CHUNK
            pltpu.make_async_copy(
                obuf.at[b], out_hbm.at[pl.ds(prev, CHUNK)], sem_s[b]).wait()

        def add_row(r, c):
            for j in range(D // 16):
                sl = pl.ds(j * 16, 16)
                obuf[b, r, sl] = sbuf[b, r, sl] + tbuf[b, r, sl]
            return c

        lax.fori_loop(0, CHUNK, add_row, 0)

        @pl.when(g + NBUF < G)
        def _():
            issue_gathers(g + NBUF, b)

        pltpu.async_copy(
            obuf.at[b], out_hbm.at[pl.ds(w_base + g * CHUNK, CHUNK)], sem_s[b])

    for b in range(NBUF):
        issue_gathers(b, b)
    for b in range(NBUF):
        handle(b, b, first=True)

    def step(i, carry):
        for b in range(NBUF):
            handle(i * NBUF + b, b, first=False)
        return carry

    lax.fori_loop(1, G // NBUF, step, 0)

    for b in range(NBUF):
        last = G - NBUF + b
        pltpu.make_async_copy(
            obuf.at[b], out_hbm.at[pl.ds(w_base + last * CHUNK, CHUNK)],
            sem_s[b]).wait()


@jax.jit
def _run(loc_3d, time_3d, spatial_table, temporal_table):
    mesh = plsc.VectorSubcoreMesh(core_axis_name="c", subcore_axis_name="s")
    f = pl.kernel(
        _sc_body,
        out_type=jax.ShapeDtypeStruct((N, D), jnp.float32),
        mesh=mesh,
        scratch_types=[
            pltpu.VMEM((G, CHUNK), jnp.int32),
            pltpu.VMEM((G, CHUNK), jnp.int32),
            pltpu.VMEM((NBUF, CHUNK, D), jnp.float32),
            pltpu.VMEM((NBUF, CHUNK, D), jnp.float32),
            pltpu.VMEM((NBUF, CHUNK, D), jnp.float32),
        ] + [pltpu.SemaphoreType.DMA] * (2 * NBUF),
        compiler_params=pltpu.CompilerParams(use_tc_tiling_on_sc=False),
    )
    return f(loc_3d, time_3d, spatial_table, temporal_table)


def kernel(loc_ids, time_ids, spatial_table, temporal_table):
    loc_3d = loc_ids.reshape(NW, G, CHUNK).astype(jnp.int32)
    time_3d = time_ids.reshape(NW, G, CHUNK).astype(jnp.int32)
    out = _run(loc_3d, time_3d, spatial_table, temporal_table)
    return out.reshape(BATCH, SEQ, D)
